# baseline (speedup 1.0000x reference)
import functools

import jax
import jax.numpy as jnp
from jax import lax
from jax.experimental import pallas as pl
from jax.experimental.pallas import tpu as pltpu
from jax.experimental.pallas import tpu_sc as plsc

KSEL = 256
BOX_HALF = 16.0
ROWS_PER_STEP = 4
LANES = 16
NSC = 2


def _matvec_body(x_ref, w_ref, o_ref):
    wv = w_ref[...]
    for r in range(x_ref.shape[0]):
        o_ref[r] = jnp.dot(wv, x_ref[r], preferred_element_type=jnp.float32)


def _matvec(x, w):
    BV, C, HW = x.shape
    R = ROWS_PER_STEP
    return pl.pallas_call(
        _matvec_body,
        grid=(BV // R,),
        in_specs=[
            pl.BlockSpec((R, C, HW), lambda i: (i, 0, 0)),
            pl.BlockSpec((1, C), lambda i: (0, 0)),
        ],
        out_specs=pl.BlockSpec((R, 1, HW), lambda i: (i, 0, 0)),
        out_shape=jax.ShapeDtypeStruct((BV, 1, HW), jnp.float32),
    )(x, w.reshape(1, C))


def _gather16(x, idx):
    dnums = lax.GatherDimensionNumbers(
        offset_dims=(), collapsed_slice_dims=(0,), start_index_map=(0,))
    return lax.gather(x, idx[:, None], dnums, (1,),
                      mode=lax.GatherScatterMode.PROMISE_IN_BOUNDS)

def _sc_select_body(scores_hbm, oval_hbm, oidx_hbm,
                    row_v, hist_v, oval_v, oidx_v):
    wid = lax.axis_index("s") * NSC + lax.axis_index("c")
    pltpu.sync_copy(scores_hbm.at[wid], row_v)

    n = row_v.shape[0]
    nstep = n // LANES
    lane = lax.iota(jnp.int32, LANES)
    ones = jnp.ones((LANES,), jnp.int32)
    zeros = jnp.zeros((LANES,), jnp.int32)

    def splat(s):
        return lax.broadcast_in_dim(s, (LANES,), ())

    prefix = zeros
    need = splat(jnp.int32(KSEL))
    for r in range(8):
        sh = 28 - 4 * r
        for bin_ in range(LANES):
            hist_v[pl.ds(bin_ * LANES, LANES)] = zeros

        def hist_step(j, carry, prefix=prefix, sh=sh, r=r):
            key = plsc.bitcast(row_v[pl.ds(j * LANES, LANES)], jnp.int32)
            digit = lax.shift_right_logical(key, sh) & 15
            slot = lane * LANES + digit
            if r == 0:
                plsc.addupdate_scatter(hist_v, [slot], ones)
            else:
                cand = (lax.shift_right_logical(key, sh + 4)
                        == lax.shift_right_logical(prefix, sh + 4))
                plsc.addupdate_scatter(hist_v, [slot], ones, mask=cand)
            return carry

        lax.fori_loop(0, nstep, hist_step, 0)

        binv = zeros
        for l in range(LANES):
            binv = binv + hist_v[pl.ds(l * LANES, LANES)]
        revc = lax.rev(binv, (0,))
        rc = plsc.cumsum(revc)
        rc_prev = rc - revc
        ge = rc >= need
        d = plsc.all_reduce_population_count(ge) - 1
        i0 = 15 - d
        above = _gather16(rc_prev, i0)
        need = need - above
        prefix = prefix | lax.shift_left(d, sh)

    tkey = prefix
    m = need

    def comp_step(j, carry):
        cgt, ceq = carry
        v = row_v[pl.ds(j * LANES, LANES)]
        key = plsc.bitcast(v, jnp.int32)
        idxv = splat(j * LANES) + lane
        gt = key > tkey
        pgt = plsc.cumsum(jnp.where(gt, 1, 0))
        pos_gt = jnp.where(gt, cgt + pgt - 1, 0)
        plsc.store_scatter(oval_v, [pos_gt], v, mask=gt)
        plsc.store_scatter(oidx_v, [pos_gt], idxv, mask=gt)
        eq = key == tkey
        peq = plsc.cumsum(jnp.where(eq, 1, 0))
        take = eq & (ceq + peq - 1 < m)
        pos_eq = jnp.where(take, splat(jnp.int32(KSEL)) - m + ceq + peq - 1, 0)
        plsc.store_scatter(oval_v, [pos_eq], v, mask=take)
        plsc.store_scatter(oidx_v, [pos_eq], idxv, mask=take)
        cgt = cgt + plsc.all_reduce_population_count(gt)
        ceq = ceq + plsc.all_reduce_population_count(eq)
        return cgt, ceq

    lax.fori_loop(0, nstep, comp_step, (zeros, zeros))

    pltpu.sync_copy(oval_v, oval_hbm.at[wid])
    pltpu.sync_copy(oidx_v, oidx_hbm.at[wid])


def _sc_select(scores):
    BV, HW = scores.shape
    mesh = plsc.VectorSubcoreMesh(core_axis_name="c", subcore_axis_name="s")
    run = functools.partial(
        pl.kernel,
        mesh=mesh,
        compiler_params=pltpu.CompilerParams(needs_layout_passes=False),
        out_type=[
            jax.ShapeDtypeStruct((BV, KSEL), jnp.float32),
            jax.ShapeDtypeStruct((BV, KSEL), jnp.int32),
        ],
        scratch_types=[
            pltpu.VMEM((HW,), jnp.float32),
            pltpu.VMEM((LANES * LANES,), jnp.int32),
            pltpu.VMEM((KSEL,), jnp.float32),
            pltpu.VMEM((KSEL,), jnp.int32),
        ],
    )(_sc_select_body)
    return run(scores)


def _rank_body(v_ref, i_ref, sy_ref, sx_ref,
               tv_ref, x1_ref, y1_ref, x2_ref, y2_ref):
    v = v_ref[0]
    vi = i_ref[0]
    vt = jnp.swapaxes(v, 0, 1)
    it = jnp.swapaxes(vi, 0, 1)
    before = (vt > v) | ((vt == v) & (it < vi))
    mat = jnp.where(before, 1.0, 0.0)
    rank = jnp.sum(mat, axis=0, keepdims=True)
    cols = lax.broadcasted_iota(jnp.int32, (1, KSEL), 1).astype(jnp.float32)
    perm = jnp.where(jnp.swapaxes(rank, 0, 1) == cols, 1.0, 0.0)
    sv = jnp.dot(v, perm, preferred_element_type=jnp.float32,
                 precision=lax.Precision.HIGHEST)
    sif = jnp.dot(vi.astype(jnp.float32), perm,
                  preferred_element_type=jnp.float32,
                  precision=lax.Precision.HIGHEST)
    iy = jnp.floor(sif * (1.0 / 64.0))
    ix = sif - iy * 64.0
    ys = iy * sy_ref[0, 0]
    xs = ix * sx_ref[0, 0]
    tv_ref[0] = sv
    x1_ref[0] = xs - BOX_HALF
    y1_ref[0] = ys - BOX_HALF
    x2_ref[0] = xs + BOX_HALF
    y2_ref[0] = ys + BOX_HALF


def _rank_sort(vals, idx, sy, sx):
    BV = vals.shape[0]
    v3 = vals.reshape(BV, 1, KSEL)
    i3 = idx.reshape(BV, 1, KSEL)
    blk = pl.BlockSpec((1, 1, KSEL), lambda i: (i, 0, 0))
    scalar = pl.BlockSpec(memory_space=pltpu.SMEM)
    outs = pl.pallas_call(
        _rank_body,
        grid=(BV,),
        in_specs=[blk, blk, scalar, scalar],
        out_specs=[blk] * 5,
        out_shape=[jax.ShapeDtypeStruct((BV, 1, KSEL), jnp.float32)] * 5,
    )(v3, i3, sy.reshape(1, 1), sx.reshape(1, 1))
    return outs


def kernel(f8, w, b, image_height, image_width):
    B, V, C, H, W = f8.shape
    HW = H * W
    BV = B * V
    x = f8.reshape(BV, C, HW)

    logits = _matvec(x, w)
    scores = jax.nn.sigmoid(logits.reshape(BV, HW) + b)

    sel_vals, sel_idx = _sc_select(scores)

    sy = (jnp.float32(image_height) / H).astype(jnp.float32)
    sx = (jnp.float32(image_width) / W).astype(jnp.float32)
    tv, x1, y1, x2, y2 = _rank_sort(sel_vals, sel_idx, sy, sx)

    top_values = tv.reshape(B, V, KSEL)
    boxes = jnp.stack(
        (x1.reshape(B, V, KSEL), y1.reshape(B, V, KSEL),
         x2.reshape(B, V, KSEL), y2.reshape(B, V, KSEL)), axis=-1)
    return boxes, top_values


# speedup vs baseline: 1.0362x; 1.0362x over previous
import functools

import jax
import jax.numpy as jnp
from jax import lax
from jax.experimental import pallas as pl
from jax.experimental.pallas import tpu as pltpu
from jax.experimental.pallas import tpu_sc as plsc

KSEL = 256
BOX_HALF = 16.0
ROWS_PER_STEP = 4
LANES = 16
NSC = 2


def _matvec_body(x_ref, w_ref, b_ref, o_ref):
    wv = w_ref[...]
    bs = b_ref[0, 0]
    for r in range(x_ref.shape[0]):
        z = jnp.dot(wv, x_ref[r], preferred_element_type=jnp.float32)
        o_ref[r] = jax.nn.sigmoid(z + bs)


def _matvec(x, w, b):
    BV, C, HW = x.shape
    R = ROWS_PER_STEP
    return pl.pallas_call(
        _matvec_body,
        grid=(BV // R,),
        in_specs=[
            pl.BlockSpec((R, C, HW), lambda i: (i, 0, 0)),
            pl.BlockSpec((1, C), lambda i: (0, 0)),
            pl.BlockSpec(memory_space=pltpu.SMEM),
        ],
        out_specs=pl.BlockSpec((R, 1, HW), lambda i: (i, 0, 0)),
        out_shape=jax.ShapeDtypeStruct((BV, 1, HW), jnp.float32),
    )(x, w.reshape(1, C), b.reshape(1, 1))


def _gather16(x, idx):
    dnums = lax.GatherDimensionNumbers(
        offset_dims=(), collapsed_slice_dims=(0,), start_index_map=(0,))
    return lax.gather(x, idx[:, None], dnums, (1,),
                      mode=lax.GatherScatterMode.PROMISE_IN_BOUNDS)

def _sc_select_body(scores_hbm, oval_hbm, oidx_hbm,
                    row_v, hist_v, oval_v, oidx_v):
    wid = lax.axis_index("s") * NSC + lax.axis_index("c")
    pltpu.sync_copy(scores_hbm.at[wid], row_v)

    n = row_v.shape[0]
    nstep = n // LANES
    lane = lax.iota(jnp.int32, LANES)
    ones = jnp.ones((LANES,), jnp.int32)
    zeros = jnp.zeros((LANES,), jnp.int32)

    def splat(s):
        return lax.broadcast_in_dim(s, (LANES,), ())

    prefix = zeros
    need = splat(jnp.int32(KSEL))
    for r in range(8):
        sh = 28 - 4 * r
        for bin_ in range(LANES):
            hist_v[pl.ds(bin_ * LANES, LANES)] = zeros

        def hist_step(j, carry, prefix=prefix, sh=sh, r=r):
            key = plsc.bitcast(row_v[pl.ds(j * LANES, LANES)], jnp.int32)
            digit = lax.shift_right_logical(key, sh) & 15
            slot = lane * LANES + digit
            if r == 0:
                plsc.addupdate_scatter(hist_v, [slot], ones)
            else:
                cand = (lax.shift_right_logical(key, sh + 4)
                        == lax.shift_right_logical(prefix, sh + 4))
                plsc.addupdate_scatter(hist_v, [slot], ones, mask=cand)
            return carry

        lax.fori_loop(0, nstep, hist_step, 0)

        binv = zeros
        for l in range(LANES):
            binv = binv + hist_v[pl.ds(l * LANES, LANES)]
        revc = lax.rev(binv, (0,))
        rc = plsc.cumsum(revc)
        rc_prev = rc - revc
        ge = rc >= need
        d = plsc.all_reduce_population_count(ge) - 1
        i0 = 15 - d
        above = _gather16(rc_prev, i0)
        need = need - above
        prefix = prefix | lax.shift_left(d, sh)

    tkey = prefix
    m = need

    def comp_step(j, carry):
        cgt, ceq = carry
        v = row_v[pl.ds(j * LANES, LANES)]
        key = plsc.bitcast(v, jnp.int32)
        idxv = splat(j * LANES) + lane
        gt = key > tkey
        pgt = plsc.cumsum(jnp.where(gt, 1, 0))
        pos_gt = jnp.where(gt, cgt + pgt - 1, 0)
        plsc.store_scatter(oval_v, [pos_gt], v, mask=gt)
        plsc.store_scatter(oidx_v, [pos_gt], idxv, mask=gt)
        eq = key == tkey
        peq = plsc.cumsum(jnp.where(eq, 1, 0))
        take = eq & (ceq + peq - 1 < m)
        pos_eq = jnp.where(take, splat(jnp.int32(KSEL)) - m + ceq + peq - 1, 0)
        plsc.store_scatter(oval_v, [pos_eq], v, mask=take)
        plsc.store_scatter(oidx_v, [pos_eq], idxv, mask=take)
        cgt = cgt + plsc.all_reduce_population_count(gt)
        ceq = ceq + plsc.all_reduce_population_count(eq)
        return cgt, ceq

    lax.fori_loop(0, nstep, comp_step, (zeros, zeros))

    pltpu.sync_copy(oval_v, oval_hbm.at[wid])
    pltpu.sync_copy(oidx_v, oidx_hbm.at[wid])


def _sc_select(scores):
    BV, HW = scores.shape
    mesh = plsc.VectorSubcoreMesh(core_axis_name="c", subcore_axis_name="s")
    run = functools.partial(
        pl.kernel,
        mesh=mesh,
        compiler_params=pltpu.CompilerParams(needs_layout_passes=False),
        out_type=[
            jax.ShapeDtypeStruct((BV, KSEL), jnp.float32),
            jax.ShapeDtypeStruct((BV, KSEL), jnp.int32),
        ],
        scratch_types=[
            pltpu.VMEM((HW,), jnp.float32),
            pltpu.VMEM((LANES * LANES,), jnp.int32),
            pltpu.VMEM((KSEL,), jnp.float32),
            pltpu.VMEM((KSEL,), jnp.int32),
        ],
    )(_sc_select_body)
    return run(scores)


def _rank_body(v_ref, i_ref, sy_ref, sx_ref,
               tv_ref, x1_ref, y1_ref, x2_ref, y2_ref):
    v = v_ref[0]
    vi = i_ref[0]
    vt = jnp.swapaxes(v, 0, 1)
    it = jnp.swapaxes(vi, 0, 1)
    before = (vt > v) | ((vt == v) & (it < vi))
    mat = jnp.where(before, 1.0, 0.0)
    rank = jnp.sum(mat, axis=0, keepdims=True)
    cols = lax.broadcasted_iota(jnp.int32, (1, KSEL), 1).astype(jnp.float32)
    pbool = jnp.swapaxes(rank, 0, 1) == cols
    sv = jnp.sum(jnp.where(pbool, vt, 0.0), axis=0, keepdims=True)
    sif = jnp.sum(jnp.where(pbool, it.astype(jnp.float32), 0.0),
                  axis=0, keepdims=True)
    iy = jnp.floor(sif * (1.0 / 64.0))
    ix = sif - iy * 64.0
    ys = iy * sy_ref[0, 0]
    xs = ix * sx_ref[0, 0]
    tv_ref[0] = sv
    x1_ref[0] = xs - BOX_HALF
    y1_ref[0] = ys - BOX_HALF
    x2_ref[0] = xs + BOX_HALF
    y2_ref[0] = ys + BOX_HALF


def _rank_sort(vals, idx, sy, sx):
    BV = vals.shape[0]
    v3 = vals.reshape(BV, 1, KSEL)
    i3 = idx.reshape(BV, 1, KSEL)
    blk = pl.BlockSpec((1, 1, KSEL), lambda i: (i, 0, 0))
    scalar = pl.BlockSpec(memory_space=pltpu.SMEM)
    outs = pl.pallas_call(
        _rank_body,
        grid=(BV,),
        in_specs=[blk, blk, scalar, scalar],
        out_specs=[blk] * 5,
        out_shape=[jax.ShapeDtypeStruct((BV, 1, KSEL), jnp.float32)] * 5,
    )(v3, i3, sy.reshape(1, 1), sx.reshape(1, 1))
    return outs


def kernel(f8, w, b, image_height, image_width):
    B, V, C, H, W = f8.shape
    HW = H * W
    BV = B * V
    x = f8.reshape(BV, C, HW)

    scores = _matvec(x, w, b.astype(jnp.float32)).reshape(BV, HW)

    sel_vals, sel_idx = _sc_select(scores)

    sy = (jnp.float32(image_height) / H).astype(jnp.float32)
    sx = (jnp.float32(image_width) / W).astype(jnp.float32)
    tv, x1, y1, x2, y2 = _rank_sort(sel_vals, sel_idx, sy, sx)

    top_values = tv.reshape(B, V, KSEL)
    boxes = jnp.stack(
        (x1.reshape(B, V, KSEL), y1.reshape(B, V, KSEL),
         x2.reshape(B, V, KSEL), y2.reshape(B, V, KSEL)), axis=-1)
    return boxes, top_values


# speedup vs baseline: 1.0973x; 1.0590x over previous
import functools

import jax
import jax.numpy as jnp
from jax import lax
from jax.experimental import pallas as pl
from jax.experimental.pallas import tpu as pltpu
from jax.experimental.pallas import tpu_sc as plsc

KSEL = 256
BOX_HALF = 16.0
ROWS_PER_STEP = 4
LANES = 16
NSC = 2


def _matvec_body(x_ref, w_ref, b_ref, o_ref):
    wv = w_ref[...]
    bs = b_ref[0, 0]
    for r in range(x_ref.shape[0]):
        z = jnp.dot(wv, x_ref[r], preferred_element_type=jnp.float32)
        o_ref[r] = jax.nn.sigmoid(z + bs)


def _matvec(x, w, b):
    BV, C, HW = x.shape
    R = ROWS_PER_STEP
    return pl.pallas_call(
        _matvec_body,
        grid=(BV // R,),
        in_specs=[
            pl.BlockSpec((R, C, HW), lambda i: (i, 0, 0)),
            pl.BlockSpec((1, C), lambda i: (0, 0)),
            pl.BlockSpec(memory_space=pltpu.SMEM),
        ],
        out_specs=pl.BlockSpec((R, 1, HW), lambda i: (i, 0, 0)),
        out_shape=jax.ShapeDtypeStruct((BV, 1, HW), jnp.float32),
    )(x, w.reshape(1, C), b.reshape(1, 1))


def _gather16(x, idx):
    dnums = lax.GatherDimensionNumbers(
        offset_dims=(), collapsed_slice_dims=(0,), start_index_map=(0,))
    return lax.gather(x, idx[:, None], dnums, (1,),
                      mode=lax.GatherScatterMode.PROMISE_IN_BOUNDS)

def _sc_select_body(scores_hbm, oval_hbm, oidx_hbm,
                    row_v, hist_v, oval_v, oidx_v):
    wid = lax.axis_index("s") * NSC + lax.axis_index("c")
    pltpu.sync_copy(scores_hbm.at[wid], row_v)

    n = row_v.shape[0]
    nstep = n // LANES
    lane = lax.iota(jnp.int32, LANES)
    ones = jnp.ones((LANES,), jnp.int32)
    zeros = jnp.zeros((LANES,), jnp.int32)

    def splat(s):
        return lax.broadcast_in_dim(s, (LANES,), ())

    prefix = zeros
    need = splat(jnp.int32(KSEL))
    for r in range(8):
        sh = 28 - 4 * r
        for bin_ in range(LANES):
            hist_v[pl.ds(bin_ * LANES, LANES)] = zeros

        def hist_step(jj, carry, prefix=prefix, sh=sh, r=r):
            for u in range(4):
                j = jj * 4 + u
                key = plsc.bitcast(row_v[pl.ds(j * LANES, LANES)], jnp.int32)
                digit = lax.shift_right_logical(key, sh) & 15
                slot = lane * LANES + digit
                if r == 0:
                    plsc.addupdate_scatter(hist_v, [slot], ones)
                else:
                    cand = (lax.shift_right_logical(key, sh + 4)
                            == lax.shift_right_logical(prefix, sh + 4))
                    plsc.addupdate_scatter(hist_v, [slot], ones, mask=cand)
            return carry

        lax.fori_loop(0, nstep // 4, hist_step, 0)

        binv = zeros
        for l in range(LANES):
            binv = binv + hist_v[pl.ds(l * LANES, LANES)]
        revc = lax.rev(binv, (0,))
        rc = plsc.cumsum(revc)
        rc_prev = rc - revc
        ge = rc >= need
        d = plsc.all_reduce_population_count(ge) - 1
        i0 = 15 - d
        above = _gather16(rc_prev, i0)
        need = need - above
        prefix = prefix | lax.shift_left(d, sh)

    tkey = prefix
    m = need

    def comp_step(j, carry):
        cgt, ceq = carry
        v = row_v[pl.ds(j * LANES, LANES)]
        key = plsc.bitcast(v, jnp.int32)
        idxv = splat(j * LANES) + lane
        gt = key > tkey
        pgt = plsc.cumsum(jnp.where(gt, 1, 0))
        pos_gt = jnp.where(gt, cgt + pgt - 1, 0)
        plsc.store_scatter(oval_v, [pos_gt], v, mask=gt)
        plsc.store_scatter(oidx_v, [pos_gt], idxv, mask=gt)
        eq = key == tkey
        peq = plsc.cumsum(jnp.where(eq, 1, 0))
        take = eq & (ceq + peq - 1 < m)
        pos_eq = jnp.where(take, splat(jnp.int32(KSEL)) - m + ceq + peq - 1, 0)
        plsc.store_scatter(oval_v, [pos_eq], v, mask=take)
        plsc.store_scatter(oidx_v, [pos_eq], idxv, mask=take)
        cgt = cgt + plsc.all_reduce_population_count(gt)
        ceq = ceq + plsc.all_reduce_population_count(eq)
        return cgt, ceq

    lax.fori_loop(0, nstep, comp_step, (zeros, zeros))

    pltpu.sync_copy(oval_v, oval_hbm.at[wid])
    pltpu.sync_copy(oidx_v, oidx_hbm.at[wid])


def _sc_select(scores):
    BV, HW = scores.shape
    mesh = plsc.VectorSubcoreMesh(core_axis_name="c", subcore_axis_name="s")
    run = functools.partial(
        pl.kernel,
        mesh=mesh,
        compiler_params=pltpu.CompilerParams(needs_layout_passes=False),
        out_type=[
            jax.ShapeDtypeStruct((BV, KSEL), jnp.float32),
            jax.ShapeDtypeStruct((BV, KSEL), jnp.int32),
        ],
        scratch_types=[
            pltpu.VMEM((HW,), jnp.float32),
            pltpu.VMEM((LANES * LANES,), jnp.int32),
            pltpu.VMEM((KSEL,), jnp.float32),
            pltpu.VMEM((KSEL,), jnp.int32),
        ],
    )(_sc_select_body)
    return run(scores)


def _rank_body(v_ref, i_ref, sy_ref, sx_ref,
               tv_ref, x1_ref, y1_ref, x2_ref, y2_ref):
    for r in range(v_ref.shape[0]):
        v = v_ref[r]
        vi = i_ref[r]
        vt = jnp.swapaxes(v, 0, 1)
        it = jnp.swapaxes(vi, 0, 1)
        before = (vt > v) | ((vt == v) & (it < vi))
        mat = jnp.where(before, 1.0, 0.0)
        rank = jnp.sum(mat, axis=0, keepdims=True)
        cols = lax.broadcasted_iota(jnp.int32, (1, KSEL), 1).astype(jnp.float32)
        pbool = jnp.swapaxes(rank, 0, 1) == cols
        sv = jnp.sum(jnp.where(pbool, vt, 0.0), axis=0, keepdims=True)
        sif = jnp.sum(jnp.where(pbool, it.astype(jnp.float32), 0.0),
                      axis=0, keepdims=True)
        iy = jnp.floor(sif * (1.0 / 64.0))
        ix = sif - iy * 64.0
        ys = iy * sy_ref[0, 0]
        xs = ix * sx_ref[0, 0]
        tv_ref[r] = sv
        x1_ref[r] = xs - BOX_HALF
        y1_ref[r] = ys - BOX_HALF
        x2_ref[r] = xs + BOX_HALF
        y2_ref[r] = ys + BOX_HALF


def _rank_sort(vals, idx, sy, sx):
    BV = vals.shape[0]
    v3 = vals.reshape(BV, 1, KSEL)
    i3 = idx.reshape(BV, 1, KSEL)
    RR = 4
    blk = pl.BlockSpec((RR, 1, KSEL), lambda i: (i, 0, 0))
    scalar = pl.BlockSpec(memory_space=pltpu.SMEM)
    outs = pl.pallas_call(
        _rank_body,
        grid=(BV // RR,),
        in_specs=[blk, blk, scalar, scalar],
        out_specs=[blk] * 5,
        out_shape=[jax.ShapeDtypeStruct((BV, 1, KSEL), jnp.float32)] * 5,
    )(v3, i3, sy.reshape(1, 1), sx.reshape(1, 1))
    return outs


def kernel(f8, w, b, image_height, image_width):
    B, V, C, H, W = f8.shape
    HW = H * W
    BV = B * V
    x = f8.reshape(BV, C, HW)

    scores = _matvec(x, w, b.astype(jnp.float32)).reshape(BV, HW)

    sel_vals, sel_idx = _sc_select(scores)

    sy = (jnp.float32(image_height) / H).astype(jnp.float32)
    sx = (jnp.float32(image_width) / W).astype(jnp.float32)
    tv, x1, y1, x2, y2 = _rank_sort(sel_vals, sel_idx, sy, sx)

    top_values = tv.reshape(B, V, KSEL)
    boxes = jnp.stack(
        (x1.reshape(B, V, KSEL), y1.reshape(B, V, KSEL),
         x2.reshape(B, V, KSEL), y2.reshape(B, V, KSEL)), axis=-1)
    return boxes, top_values
